# baseline (speedup 1.0000x reference)
import functools
import jax
import jax.numpy as jnp
from jax import lax
from jax.experimental import pallas as pl
from jax.experimental.pallas import tpu as pltpu
from jax.experimental.pallas import tpu_sc as plsc

NEG = 0.2
NN = 10000
EE = 160000
LL = 16
NC = 2
NS = 16
NP = 10240
ROWS_PER_TEC = NP // NS


def _lrelu(x):
    return jnp.where(x > 0, x, NEG * x)


def _tc1_body(x_ref, w1_ref, as_m_ref, ad_m_ref,
              h_ref, as_ref, ad_ref, ms_ref, md_ref):
    i = pl.program_id(0)
    h = jnp.dot(x_ref[...], w1_ref[...], preferred_element_type=jnp.float32)
    h_ref[...] = h
    a_s = jnp.dot(h, as_m_ref[...], preferred_element_type=jnp.float32)
    a_d = jnp.dot(h, ad_m_ref[...], preferred_element_type=jnp.float32)
    as_ref[...] = a_s
    ad_ref[...] = a_d

    @pl.when(i == 0)
    def _():
        ms_ref[...] = jnp.full((1, LL), -1e30, jnp.float32)
        md_ref[...] = jnp.full((1, LL), -1e30, jnp.float32)

    ms_ref[...] = jnp.maximum(ms_ref[...], jnp.max(a_s, axis=0, keepdims=True))
    md_ref[...] = jnp.maximum(md_ref[...], jnp.max(a_d, axis=0, keepdims=True))


def _tc1(x, W1, As_m, Ad_m, block=1000):
    g = NN // block
    return pl.pallas_call(
        _tc1_body,
        grid=(g,),
        in_specs=[
            pl.BlockSpec((block, 128), lambda i: (i, 0)),
            pl.BlockSpec((128, 320), lambda i: (0, 0)),
            pl.BlockSpec((320, LL), lambda i: (0, 0)),
            pl.BlockSpec((320, LL), lambda i: (0, 0)),
        ],
        out_specs=[
            pl.BlockSpec((block, 320), lambda i: (i, 0)),
            pl.BlockSpec((block, LL), lambda i: (i, 0)),
            pl.BlockSpec((block, LL), lambda i: (i, 0)),
            pl.BlockSpec((1, LL), lambda i: (0, 0)),
            pl.BlockSpec((1, LL), lambda i: (0, 0)),
        ],
        out_shape=[
            jax.ShapeDtypeStruct((NN, 320), jnp.float32),
            jax.ShapeDtypeStruct((NN, LL), jnp.float32),
            jax.ShapeDtypeStruct((NN, LL), jnp.float32),
            jax.ShapeDtypeStruct((1, LL), jnp.float32),
            jax.ShapeDtypeStruct((1, LL), jnp.float32),
        ],
    )(x, W1, As_m, Ad_m)


def _subchunks(B):
    subs, off = [], 0
    while off < B:
        sz = min(128, B - off)
        subs.append((off, sz))
        off += sz
    return subs


def _sc_edge_body(D, HLOC, B, CHUNKS,
                  hcat, srcA, srcB, dstA, dstB, as_cat, ad_cat, msmd, zmsg,
                  zden,
                  msg_out, den_out,
                  msg_sh, den_sh,
                  src_v, srch_v, dsth_v, rows_v, as_v, ad_v, w_v,
                  msmd_v, dst_subs, sem1, sem2, sem3):
    c = lax.axis_index("c")
    s = lax.axis_index("s")
    subs = _subchunks(B)

    r0 = s * ROWS_PER_TEC
    pltpu.sync_copy(zmsg, msg_sh.at[pl.ds(r0, ROWS_PER_TEC)])
    pltpu.sync_copy(zden, den_sh.at[pl.ds(r0, ROWS_PER_TEC)])

    pltpu.sync_copy(msmd, msmd_v)
    csum0 = msmd_v[0, :] + msmd_v[2, :]
    csum1 = msmd_v[1, :] + msmd_v[3, :]
    csum = jnp.where(c == 0, csum0, csum1)
    cvec = jnp.maximum(_lrelu(csum), 0.0)

    plsc.subcore_barrier()

    if HLOC == 5:
        tec_edges = EE // NS
        edge_base0 = s * tec_edges
    else:
        tec_edges = EE // (NC * NS)
        edge_base0 = c * (EE // NC) + s * tec_edges

    boff = c * EE

    def chunk_body(k, _):
        base = edge_base0 + k * B
        pltpu.sync_copy(srcA.at[pl.ds(base, B)], src_v)
        pltpu.sync_copy(srcB.at[pl.ds(boff + base, B)], srch_v)
        pltpu.sync_copy(dstB.at[pl.ds(boff + base, B)], dsth_v)
        for t, (off, sz) in enumerate(subs):
            pltpu.sync_copy(dstA.at[pl.ds(base + off, sz)], dst_subs[t])
        hidx = srch_v if HLOC == 5 else src_v
        cps = []
        for off, sz in subs:
            cps.append(pltpu.async_copy(
                hcat.at[hidx.at[pl.ds(off, sz)]],
                rows_v.at[pl.ds(off, sz)], sem1))
            cps.append(pltpu.async_copy(
                as_cat.at[srch_v.at[pl.ds(off, sz)]],
                as_v.at[pl.ds(off, sz)], sem2))
            cps.append(pltpu.async_copy(
                ad_cat.at[dsth_v.at[pl.ds(off, sz)]],
                ad_v.at[pl.ds(off, sz)], sem3))
        for cp in cps:
            cp.wait()

        def edge_body(i, _):
            lg = as_v[i, :] + ad_v[i, :]
            w = jnp.exp(_lrelu(lg) - cvec)
            w_v[i, :] = w
            for hh in range(HLOC):
                wsc = w[hh]
                for q in range(2):
                    off = hh * 32 + q * LL
                    rows_v[i, pl.ds(off, LL)] = rows_v[i, pl.ds(off, LL)] * wsc
            return 0

        lax.fori_loop(0, B, edge_body, 0, unroll=4)
        for t, (off, sz) in enumerate(subs):
            pltpu.sync_copy(w_v.at[pl.ds(off, sz)],
                            den_sh.at[dst_subs[t]], add=True)
            pltpu.sync_copy(rows_v.at[pl.ds(off, sz)],
                            msg_sh.at[dst_subs[t]], add=True)
        return 0

    lax.fori_loop(0, CHUNKS, chunk_body, 0)

    plsc.subcore_barrier()

    orow = c * NP + r0
    pltpu.sync_copy(msg_sh.at[pl.ds(r0, ROWS_PER_TEC)],
                    msg_out.at[pl.ds(orow, ROWS_PER_TEC)])
    pltpu.sync_copy(den_sh.at[pl.ds(r0, ROWS_PER_TEC)],
                    den_out.at[pl.ds(orow, ROWS_PER_TEC)])


def _sc_edge(hcat, srcA, srcB, dstA, dstB, as_cat, ad_cat, msmd,
             D, HLOC, B, CHUNKS):
    mesh = plsc.VectorSubcoreMesh(core_axis_name="c", subcore_axis_name="s")
    zmsg = jnp.zeros((ROWS_PER_TEC, D), jnp.float32)
    zden = jnp.zeros((ROWS_PER_TEC, LL), jnp.float32)
    body = functools.partial(_sc_edge_body, D, HLOC, B, CHUNKS)
    fn = pl.kernel(
        body,
        out_type=[
            jax.ShapeDtypeStruct((NC * NP, D), jnp.float32),
            jax.ShapeDtypeStruct((NC * NP, LL), jnp.float32),
        ],
        mesh=mesh,
        scratch_types=[
            pltpu.VMEM_SHARED((NP, D), jnp.float32),
            pltpu.VMEM_SHARED((NP, LL), jnp.float32),
            pltpu.VMEM((B,), jnp.int32),
            pltpu.VMEM((B,), jnp.int32),
            pltpu.VMEM((B,), jnp.int32),
            pltpu.VMEM((B, D), jnp.float32),
            pltpu.VMEM((B, LL), jnp.float32),
            pltpu.VMEM((B, LL), jnp.float32),
            pltpu.VMEM((B, LL), jnp.float32),
            pltpu.VMEM((4, LL), jnp.float32),
            [pltpu.VMEM((sz,), jnp.int32) for _, sz in _subchunks(B)],
            pltpu.SemaphoreType.DMA,
            pltpu.SemaphoreType.DMA,
            pltpu.SemaphoreType.DMA,
        ],
        compiler_params=pltpu.CompilerParams(use_tc_tiling_on_sc=False),
    )
    return fn(hcat, srcA, srcB, dstA, dstB, as_cat, ad_cat, msmd, zmsg, zden)


def _tc2_body(msg_ref, den_ref, h_ref, as_ref, ad_ref, ms_ref, md_ref,
              b1_ref, w2_ref, as2m_ref, ad2m_ref, r1_ref,
              h2_ref, as2_ref, ad2_ref, ms2_ref, md2_ref):
    i = pl.program_id(0)
    csum = ms_ref[...] + md_ref[...]
    c1 = jnp.maximum(_lrelu(csum), 0.0)
    sl = _lrelu(as_ref[...] + ad_ref[...])
    wself = jnp.exp(sl - c1)
    dt = den_ref[...] + wself
    wb = jnp.dot(wself, r1_ref[...], preferred_element_type=jnp.float32)
    db = jnp.dot(dt, r1_ref[...], preferred_element_type=jnp.float32)
    msgt = msg_ref[...] + h_ref[...] * wb
    o1 = msgt / jnp.clip(db, 1e-16) + b1_ref[...]
    h1 = jnp.where(o1 > 0, o1, jnp.exp(jnp.minimum(o1, 0.0)) - 1.0)
    h2p = jnp.dot(h1, w2_ref[...], preferred_element_type=jnp.float32)
    h2_ref[...] = h2p
    a_s2 = jnp.dot(h2p, as2m_ref[...], preferred_element_type=jnp.float32)
    a_d2 = jnp.dot(h2p, ad2m_ref[...], preferred_element_type=jnp.float32)
    as2_ref[...] = a_s2
    ad2_ref[...] = a_d2

    @pl.when(i == 0)
    def _():
        ms2_ref[...] = jnp.full((1, LL), -1e30, jnp.float32)
        md2_ref[...] = jnp.full((1, LL), -1e30, jnp.float32)

    ms2_ref[...] = jnp.maximum(ms2_ref[...], jnp.max(a_s2, 0, keepdims=True))
    md2_ref[...] = jnp.maximum(md2_ref[...], jnp.max(a_d2, 0, keepdims=True))


def _tc2(msg1, den1, h, a_s1, a_d1, ms1, md1, b1, W2, As2_m, Ad2_m, R1,
         block=1000):
    g = NN // block
    cst = lambda i: (0, 0)
    blk = lambda i: (i, 0)
    return pl.pallas_call(
        _tc2_body,
        grid=(g,),
        in_specs=[
            pl.BlockSpec((block, 320), blk),
            pl.BlockSpec((block, LL), blk),
            pl.BlockSpec((block, 320), blk),
            pl.BlockSpec((block, LL), blk),
            pl.BlockSpec((block, LL), blk),
            pl.BlockSpec((1, LL), cst),
            pl.BlockSpec((1, LL), cst),
            pl.BlockSpec((1, 320), cst),
            pl.BlockSpec((320, 32), cst),
            pl.BlockSpec((32, LL), cst),
            pl.BlockSpec((32, LL), cst),
            pl.BlockSpec((LL, 320), cst),
        ],
        out_specs=[
            pl.BlockSpec((block, 32), blk),
            pl.BlockSpec((block, LL), blk),
            pl.BlockSpec((block, LL), blk),
            pl.BlockSpec((1, LL), cst),
            pl.BlockSpec((1, LL), cst),
        ],
        out_shape=[
            jax.ShapeDtypeStruct((NN, 32), jnp.float32),
            jax.ShapeDtypeStruct((NN, LL), jnp.float32),
            jax.ShapeDtypeStruct((NN, LL), jnp.float32),
            jax.ShapeDtypeStruct((1, LL), jnp.float32),
            jax.ShapeDtypeStruct((1, LL), jnp.float32),
        ],
    )(msg1, den1, h, a_s1, a_d1, ms1, md1, b1, W2, As2_m, Ad2_m, R1)


def _tc3_body(ma_ref, mb_ref, da_ref, db_ref, h2_ref, as_ref, ad_ref,
              ms_ref, md_ref, b2_ref, r2_ref, wfc_ref, bfc_ref, out_ref):
    i = pl.program_id(0)
    ng = pl.num_programs(0)
    csum = ms_ref[...] + md_ref[...]
    c2 = jnp.maximum(_lrelu(csum), 0.0)
    sl = _lrelu(as_ref[...] + ad_ref[...])
    wself = jnp.exp(sl - c2)
    dt = da_ref[...] + db_ref[...] + wself
    wb = jnp.dot(wself, r2_ref[...], preferred_element_type=jnp.float32)
    dbb = jnp.dot(dt, r2_ref[...], preferred_element_type=jnp.float32)
    msgt = ma_ref[...] + mb_ref[...] + h2_ref[...] * wb
    o2 = msgt / jnp.clip(dbb, 1e-16) + b2_ref[...]
    h2 = jnp.where(o2 > 0, o2, jnp.exp(jnp.minimum(o2, 0.0)) - 1.0)
    m = jnp.max(h2, axis=0, keepdims=True)

    @pl.when(i == 0)
    def _():
        out_ref[...] = jnp.full((1, 32), -1e30, jnp.float32)

    out_ref[...] = jnp.maximum(out_ref[...], m)

    @pl.when(i == ng - 1)
    def _():
        pooled = out_ref[...]
        fc = jnp.dot(pooled, wfc_ref[...],
                     preferred_element_type=jnp.float32) + bfc_ref[...]
        out_ref[...] = jnp.maximum(fc, 0.0)


def _tc3(ma, mb, da, db, h2p, as2, ad2, ms2, md2, b2, R2, Wfc, bfc,
         block=1000):
    g = NN // block
    cst = lambda i: (0, 0)
    blk = lambda i: (i, 0)
    return pl.pallas_call(
        _tc3_body,
        grid=(g,),
        in_specs=[
            pl.BlockSpec((block, 32), blk),
            pl.BlockSpec((block, 32), blk),
            pl.BlockSpec((block, LL), blk),
            pl.BlockSpec((block, LL), blk),
            pl.BlockSpec((block, 32), blk),
            pl.BlockSpec((block, LL), blk),
            pl.BlockSpec((block, LL), blk),
            pl.BlockSpec((1, LL), cst),
            pl.BlockSpec((1, LL), cst),
            pl.BlockSpec((1, 32), cst),
            pl.BlockSpec((LL, 32), cst),
            pl.BlockSpec((32, 32), cst),
            pl.BlockSpec((1, 32), cst),
        ],
        out_specs=pl.BlockSpec((1, 32), cst),
        out_shape=jax.ShapeDtypeStruct((1, 32), jnp.float32),
    )(ma, mb, da, db, h2p, as2, ad2, ms2, md2, b2, R2, Wfc, bfc)


def kernel(x, edge_index, W1, a_s1, a_d1, b1, W2, a_s2, a_d2, b2, Wfc, bfc):
    f32 = jnp.float32
    srcE = edge_index[0].astype(jnp.int32)
    dstE = edge_index[1].astype(jnp.int32)
    srcB = jnp.concatenate([srcE, srcE + NN])
    dstB = jnp.concatenate([dstE, dstE + NN])

    heads320 = jnp.repeat(jnp.arange(10, dtype=jnp.int32), 32)
    As1_m = jnp.zeros((320, LL), f32).at[jnp.arange(320), heads320].set(
        a_s1.reshape(320))
    Ad1_m = jnp.zeros((320, LL), f32).at[jnp.arange(320), heads320].set(
        a_d1.reshape(320))
    As2_m = jnp.zeros((32, LL), f32).at[:, 0].set(a_s2.reshape(32))
    Ad2_m = jnp.zeros((32, LL), f32).at[:, 0].set(a_d2.reshape(32))
    R1 = jnp.zeros((LL, 320), f32).at[heads320, jnp.arange(320)].set(1.0)
    R2 = jnp.zeros((LL, 32), f32).at[0, :].set(1.0)

    h, a_s, a_d, ms1, md1 = _tc1(x, W1, As1_m, Ad1_m)
    shift5 = lambda a: jnp.concatenate(
        [a[:, 5:], jnp.zeros((a.shape[0], 5), f32)], axis=1)
    hcat = jnp.concatenate([h[:, :160], h[:, 160:]], axis=0)
    as_cat = jnp.concatenate([a_s, shift5(a_s)], axis=0)
    ad_cat = jnp.concatenate([a_d, shift5(a_d)], axis=0)
    msmd1 = jnp.concatenate([ms1, shift5(ms1), md1, shift5(md1)], axis=0)
    msg1_2, den1_2 = _sc_edge(hcat, srcE, srcB, dstE, dstB, as_cat, ad_cat,
                              msmd1,
                              D=160, HLOC=5, B=80, CHUNKS=(EE // NS) // 80)
    msg1 = jnp.concatenate([msg1_2[:NN], msg1_2[NP:NP + NN]], axis=1)
    den1 = den1_2[:NN]

    h2p, as2v, ad2v, ms2, md2 = _tc2(
        msg1, den1, h, a_s, a_d, ms1, md1, b1.reshape(1, 320), W2,
        As2_m, Ad2_m, R1)

    as2_cat = jnp.concatenate([as2v, as2v], axis=0)
    ad2_cat = jnp.concatenate([ad2v, ad2v], axis=0)
    msmd2 = jnp.concatenate([ms2, ms2, md2, md2], axis=0)
    msg2_2, den2_2 = _sc_edge(h2p, srcE, srcB, dstE, dstB, as2_cat, ad2_cat,
                              msmd2,
                              D=32, HLOC=1,
                              B=200, CHUNKS=(EE // (NC * NS)) // 200)

    out = _tc3(msg2_2[:NN], msg2_2[NP:NP + NN], den2_2[:NN], den2_2[NP:NP + NN],
               h2p, as2v, ad2v, ms2, md2, b2.reshape(1, 32), R2, Wfc,
               bfc.reshape(1, 32))
    return out


# speedup vs baseline: 1.1902x; 1.1902x over previous
import functools
import jax
import jax.numpy as jnp
from jax import lax
from jax.experimental import pallas as pl
from jax.experimental.pallas import tpu as pltpu
from jax.experimental.pallas import tpu_sc as plsc

NEG = 0.2
NN = 10000
EE = 160000
LL = 16
NC = 2
NS = 16
NP = 10240
ROWS_PER_TEC = NP // NS


def _lrelu(x):
    return jnp.where(x > 0, x, NEG * x)


def _tc1_body(x_ref, w1_ref, as_m_ref, ad_m_ref,
              h_ref, as_ref, ad_ref, ms_ref, md_ref):
    i = pl.program_id(0)
    h = jnp.dot(x_ref[...], w1_ref[...], preferred_element_type=jnp.float32)
    h_ref[...] = h
    a_s = jnp.dot(h, as_m_ref[...], preferred_element_type=jnp.float32)
    a_d = jnp.dot(h, ad_m_ref[...], preferred_element_type=jnp.float32)
    as_ref[...] = a_s
    ad_ref[...] = a_d

    @pl.when(i == 0)
    def _():
        ms_ref[...] = jnp.full((1, LL), -1e30, jnp.float32)
        md_ref[...] = jnp.full((1, LL), -1e30, jnp.float32)

    ms_ref[...] = jnp.maximum(ms_ref[...], jnp.max(a_s, axis=0, keepdims=True))
    md_ref[...] = jnp.maximum(md_ref[...], jnp.max(a_d, axis=0, keepdims=True))


def _tc1(x, W1, As_m, Ad_m, block=1000):
    g = NN // block
    return pl.pallas_call(
        _tc1_body,
        grid=(g,),
        in_specs=[
            pl.BlockSpec((block, 128), lambda i: (i, 0)),
            pl.BlockSpec((128, 320), lambda i: (0, 0)),
            pl.BlockSpec((320, LL), lambda i: (0, 0)),
            pl.BlockSpec((320, LL), lambda i: (0, 0)),
        ],
        out_specs=[
            pl.BlockSpec((block, 320), lambda i: (i, 0)),
            pl.BlockSpec((block, LL), lambda i: (i, 0)),
            pl.BlockSpec((block, LL), lambda i: (i, 0)),
            pl.BlockSpec((1, LL), lambda i: (0, 0)),
            pl.BlockSpec((1, LL), lambda i: (0, 0)),
        ],
        out_shape=[
            jax.ShapeDtypeStruct((NN, 320), jnp.float32),
            jax.ShapeDtypeStruct((NN, LL), jnp.float32),
            jax.ShapeDtypeStruct((NN, LL), jnp.float32),
            jax.ShapeDtypeStruct((1, LL), jnp.float32),
            jax.ShapeDtypeStruct((1, LL), jnp.float32),
        ],
    )(x, W1, As_m, Ad_m)


def _subchunks(B):
    subs, off = [], 0
    while off < B:
        sz = min(128, B - off)
        subs.append((off, sz))
        off += sz
    return subs


def _sc_edge_body(D, HLOC, B, CHUNKS,
                  hcat, srcA, srcB, dstA, dstB, as_cat, ad_cat, msmd, zmsg,
                  zden,
                  msg_out, den_out,
                  msg_sh, den_sh,
                  src_v, srch_v, dsth_v, rows_v, as_v, ad_v, w_v,
                  msmd_v, dst_subs, sem1, sem2, sem3):
    c = lax.axis_index("c")
    s = lax.axis_index("s")
    subs = _subchunks(B)

    r0 = s * ROWS_PER_TEC
    pltpu.sync_copy(zmsg, msg_sh.at[pl.ds(r0, ROWS_PER_TEC)])
    pltpu.sync_copy(zden, den_sh.at[pl.ds(r0, ROWS_PER_TEC)])

    pltpu.sync_copy(msmd, msmd_v)
    csum0 = msmd_v[0, :] + msmd_v[2, :]
    csum1 = msmd_v[1, :] + msmd_v[3, :]
    csum = jnp.where(c == 0, csum0, csum1)
    cvec = jnp.maximum(_lrelu(csum), 0.0)

    plsc.subcore_barrier()

    if HLOC == 5:
        tec_edges = EE // NS
        edge_base0 = s * tec_edges
    else:
        tec_edges = EE // (NC * NS)
        edge_base0 = c * (EE // NC) + s * tec_edges

    boff = c * EE

    def load_idx(p, k):
        base = edge_base0 + k * B
        pltpu.sync_copy(srcA.at[pl.ds(base, B)], src_v[p])
        pltpu.sync_copy(srcB.at[pl.ds(boff + base, B)], srch_v[p])
        pltpu.sync_copy(dstB.at[pl.ds(boff + base, B)], dsth_v[p])
        for t, (off, sz) in enumerate(subs):
            pltpu.sync_copy(dstA.at[pl.ds(base + off, sz)], dst_subs[p][t])

    def gather_cps(p):
        hidx = srch_v[p] if HLOC == 5 else src_v[p]
        cps = []
        for off, sz in subs:
            cps.append((hcat.at[hidx.at[pl.ds(off, sz)]],
                        rows_v[p].at[pl.ds(off, sz)], sem1[p]))
            cps.append((as_cat.at[srch_v[p].at[pl.ds(off, sz)]],
                        as_v[p].at[pl.ds(off, sz)], sem2[p]))
            cps.append((ad_cat.at[dsth_v[p].at[pl.ds(off, sz)]],
                        ad_v[p].at[pl.ds(off, sz)], sem3[p]))
        return cps

    def fire(p):
        for a, b, sm in gather_cps(p):
            pltpu.async_copy(a, b, sm)

    def drain(p):
        for a, b, sm in gather_cps(p):
            pltpu.make_async_copy(a, b, sm).wait()

    def compute_scatter(p):
        def edge_body(i, _):
            lg = as_v[p][i, :] + ad_v[p][i, :]
            w = jnp.exp(_lrelu(lg) - cvec)
            w_v[p][i, :] = w
            for hh in range(HLOC):
                wsc = w[hh]
                for q in range(2):
                    off = hh * 32 + q * LL
                    rows_v[p][i, pl.ds(off, LL)] = (
                        rows_v[p][i, pl.ds(off, LL)] * wsc)
            return 0

        lax.fori_loop(0, B, edge_body, 0, unroll=2)
        for t, (off, sz) in enumerate(subs):
            pltpu.sync_copy(w_v[p].at[pl.ds(off, sz)],
                            den_sh.at[dst_subs[p][t]], add=True)
            pltpu.sync_copy(rows_v[p].at[pl.ds(off, sz)],
                            msg_sh.at[dst_subs[p][t]], add=True)

    pairs = (CHUNKS - 1) // 2
    load_idx(0, 0)
    fire(0)

    def pair_body(j, _):
        load_idx(1, 2 * j + 1)
        fire(1)
        drain(0)
        compute_scatter(0)
        load_idx(0, 2 * j + 2)
        fire(0)
        drain(1)
        compute_scatter(1)
        return 0

    lax.fori_loop(0, pairs, pair_body, 0)
    drain(0)
    compute_scatter(0)
    if CHUNKS % 2 == 0:
        load_idx(1, CHUNKS - 1)
        fire(1)
        drain(1)
        compute_scatter(1)

    plsc.subcore_barrier()

    orow = c * NP + r0
    pltpu.sync_copy(msg_sh.at[pl.ds(r0, ROWS_PER_TEC)],
                    msg_out.at[pl.ds(orow, ROWS_PER_TEC)])
    pltpu.sync_copy(den_sh.at[pl.ds(r0, ROWS_PER_TEC)],
                    den_out.at[pl.ds(orow, ROWS_PER_TEC)])


def _sc_edge(hcat, srcA, srcB, dstA, dstB, as_cat, ad_cat, msmd,
             D, HLOC, B, CHUNKS):
    mesh = plsc.VectorSubcoreMesh(core_axis_name="c", subcore_axis_name="s")
    zmsg = jnp.zeros((ROWS_PER_TEC, D), jnp.float32)
    zden = jnp.zeros((ROWS_PER_TEC, LL), jnp.float32)
    body = functools.partial(_sc_edge_body, D, HLOC, B, CHUNKS)
    fn = pl.kernel(
        body,
        out_type=[
            jax.ShapeDtypeStruct((NC * NP, D), jnp.float32),
            jax.ShapeDtypeStruct((NC * NP, LL), jnp.float32),
        ],
        mesh=mesh,
        scratch_types=[
            pltpu.VMEM_SHARED((NP, D), jnp.float32),
            pltpu.VMEM_SHARED((NP, LL), jnp.float32),
            [pltpu.VMEM((B,), jnp.int32) for _ in range(2)],
            [pltpu.VMEM((B,), jnp.int32) for _ in range(2)],
            [pltpu.VMEM((B,), jnp.int32) for _ in range(2)],
            [pltpu.VMEM((B, D), jnp.float32) for _ in range(2)],
            [pltpu.VMEM((B, LL), jnp.float32) for _ in range(2)],
            [pltpu.VMEM((B, LL), jnp.float32) for _ in range(2)],
            [pltpu.VMEM((B, LL), jnp.float32) for _ in range(2)],
            pltpu.VMEM((4, LL), jnp.float32),
            [[pltpu.VMEM((sz,), jnp.int32) for _, sz in _subchunks(B)]
             for _ in range(2)],
            [pltpu.SemaphoreType.DMA for _ in range(2)],
            [pltpu.SemaphoreType.DMA for _ in range(2)],
            [pltpu.SemaphoreType.DMA for _ in range(2)],
        ],
        compiler_params=pltpu.CompilerParams(use_tc_tiling_on_sc=False),
    )
    return fn(hcat, srcA, srcB, dstA, dstB, as_cat, ad_cat, msmd, zmsg, zden)


def _tc2_body(msg_ref, den_ref, h_ref, as_ref, ad_ref, ms_ref, md_ref,
              b1_ref, w2_ref, as2m_ref, ad2m_ref, r1_ref,
              h2_ref, as2_ref, ad2_ref, ms2_ref, md2_ref):
    i = pl.program_id(0)
    csum = ms_ref[...] + md_ref[...]
    c1 = jnp.maximum(_lrelu(csum), 0.0)
    sl = _lrelu(as_ref[...] + ad_ref[...])
    wself = jnp.exp(sl - c1)
    dt = den_ref[...] + wself
    wb = jnp.dot(wself, r1_ref[...], preferred_element_type=jnp.float32)
    db = jnp.dot(dt, r1_ref[...], preferred_element_type=jnp.float32)
    msgt = msg_ref[...] + h_ref[...] * wb
    o1 = msgt / jnp.clip(db, 1e-16) + b1_ref[...]
    h1 = jnp.where(o1 > 0, o1, jnp.exp(jnp.minimum(o1, 0.0)) - 1.0)
    h2p = jnp.dot(h1, w2_ref[...], preferred_element_type=jnp.float32)
    h2_ref[...] = h2p
    a_s2 = jnp.dot(h2p, as2m_ref[...], preferred_element_type=jnp.float32)
    a_d2 = jnp.dot(h2p, ad2m_ref[...], preferred_element_type=jnp.float32)
    as2_ref[...] = a_s2
    ad2_ref[...] = a_d2

    @pl.when(i == 0)
    def _():
        ms2_ref[...] = jnp.full((1, LL), -1e30, jnp.float32)
        md2_ref[...] = jnp.full((1, LL), -1e30, jnp.float32)

    ms2_ref[...] = jnp.maximum(ms2_ref[...], jnp.max(a_s2, 0, keepdims=True))
    md2_ref[...] = jnp.maximum(md2_ref[...], jnp.max(a_d2, 0, keepdims=True))


def _tc2(msg1, den1, h, a_s1, a_d1, ms1, md1, b1, W2, As2_m, Ad2_m, R1,
         block=1000):
    g = NN // block
    cst = lambda i: (0, 0)
    blk = lambda i: (i, 0)
    return pl.pallas_call(
        _tc2_body,
        grid=(g,),
        in_specs=[
            pl.BlockSpec((block, 320), blk),
            pl.BlockSpec((block, LL), blk),
            pl.BlockSpec((block, 320), blk),
            pl.BlockSpec((block, LL), blk),
            pl.BlockSpec((block, LL), blk),
            pl.BlockSpec((1, LL), cst),
            pl.BlockSpec((1, LL), cst),
            pl.BlockSpec((1, 320), cst),
            pl.BlockSpec((320, 32), cst),
            pl.BlockSpec((32, LL), cst),
            pl.BlockSpec((32, LL), cst),
            pl.BlockSpec((LL, 320), cst),
        ],
        out_specs=[
            pl.BlockSpec((block, 32), blk),
            pl.BlockSpec((block, LL), blk),
            pl.BlockSpec((block, LL), blk),
            pl.BlockSpec((1, LL), cst),
            pl.BlockSpec((1, LL), cst),
        ],
        out_shape=[
            jax.ShapeDtypeStruct((NN, 32), jnp.float32),
            jax.ShapeDtypeStruct((NN, LL), jnp.float32),
            jax.ShapeDtypeStruct((NN, LL), jnp.float32),
            jax.ShapeDtypeStruct((1, LL), jnp.float32),
            jax.ShapeDtypeStruct((1, LL), jnp.float32),
        ],
    )(msg1, den1, h, a_s1, a_d1, ms1, md1, b1, W2, As2_m, Ad2_m, R1)


def _tc3_body(ma_ref, mb_ref, da_ref, db_ref, h2_ref, as_ref, ad_ref,
              ms_ref, md_ref, b2_ref, r2_ref, wfc_ref, bfc_ref, out_ref):
    i = pl.program_id(0)
    ng = pl.num_programs(0)
    csum = ms_ref[...] + md_ref[...]
    c2 = jnp.maximum(_lrelu(csum), 0.0)
    sl = _lrelu(as_ref[...] + ad_ref[...])
    wself = jnp.exp(sl - c2)
    dt = da_ref[...] + db_ref[...] + wself
    wb = jnp.dot(wself, r2_ref[...], preferred_element_type=jnp.float32)
    dbb = jnp.dot(dt, r2_ref[...], preferred_element_type=jnp.float32)
    msgt = ma_ref[...] + mb_ref[...] + h2_ref[...] * wb
    o2 = msgt / jnp.clip(dbb, 1e-16) + b2_ref[...]
    h2 = jnp.where(o2 > 0, o2, jnp.exp(jnp.minimum(o2, 0.0)) - 1.0)
    m = jnp.max(h2, axis=0, keepdims=True)

    @pl.when(i == 0)
    def _():
        out_ref[...] = jnp.full((1, 32), -1e30, jnp.float32)

    out_ref[...] = jnp.maximum(out_ref[...], m)

    @pl.when(i == ng - 1)
    def _():
        pooled = out_ref[...]
        fc = jnp.dot(pooled, wfc_ref[...],
                     preferred_element_type=jnp.float32) + bfc_ref[...]
        out_ref[...] = jnp.maximum(fc, 0.0)


def _tc3(ma, mb, da, db, h2p, as2, ad2, ms2, md2, b2, R2, Wfc, bfc,
         block=1000):
    g = NN // block
    cst = lambda i: (0, 0)
    blk = lambda i: (i, 0)
    return pl.pallas_call(
        _tc3_body,
        grid=(g,),
        in_specs=[
            pl.BlockSpec((block, 32), blk),
            pl.BlockSpec((block, 32), blk),
            pl.BlockSpec((block, LL), blk),
            pl.BlockSpec((block, LL), blk),
            pl.BlockSpec((block, 32), blk),
            pl.BlockSpec((block, LL), blk),
            pl.BlockSpec((block, LL), blk),
            pl.BlockSpec((1, LL), cst),
            pl.BlockSpec((1, LL), cst),
            pl.BlockSpec((1, 32), cst),
            pl.BlockSpec((LL, 32), cst),
            pl.BlockSpec((32, 32), cst),
            pl.BlockSpec((1, 32), cst),
        ],
        out_specs=pl.BlockSpec((1, 32), cst),
        out_shape=jax.ShapeDtypeStruct((1, 32), jnp.float32),
    )(ma, mb, da, db, h2p, as2, ad2, ms2, md2, b2, R2, Wfc, bfc)


def kernel(x, edge_index, W1, a_s1, a_d1, b1, W2, a_s2, a_d2, b2, Wfc, bfc):
    f32 = jnp.float32
    srcE = edge_index[0].astype(jnp.int32)
    dstE = edge_index[1].astype(jnp.int32)
    srcB = jnp.concatenate([srcE, srcE + NN])
    dstB = jnp.concatenate([dstE, dstE + NN])

    heads320 = jnp.repeat(jnp.arange(10, dtype=jnp.int32), 32)
    As1_m = jnp.zeros((320, LL), f32).at[jnp.arange(320), heads320].set(
        a_s1.reshape(320))
    Ad1_m = jnp.zeros((320, LL), f32).at[jnp.arange(320), heads320].set(
        a_d1.reshape(320))
    As2_m = jnp.zeros((32, LL), f32).at[:, 0].set(a_s2.reshape(32))
    Ad2_m = jnp.zeros((32, LL), f32).at[:, 0].set(a_d2.reshape(32))
    R1 = jnp.zeros((LL, 320), f32).at[heads320, jnp.arange(320)].set(1.0)
    R2 = jnp.zeros((LL, 32), f32).at[0, :].set(1.0)

    h, a_s, a_d, ms1, md1 = _tc1(x, W1, As1_m, Ad1_m)
    shift5 = lambda a: jnp.concatenate(
        [a[:, 5:], jnp.zeros((a.shape[0], 5), f32)], axis=1)
    hcat = jnp.concatenate([h[:, :160], h[:, 160:]], axis=0)
    as_cat = jnp.concatenate([a_s, shift5(a_s)], axis=0)
    ad_cat = jnp.concatenate([a_d, shift5(a_d)], axis=0)
    msmd1 = jnp.concatenate([ms1, shift5(ms1), md1, shift5(md1)], axis=0)
    msg1_2, den1_2 = _sc_edge(hcat, srcE, srcB, dstE, dstB, as_cat, ad_cat,
                              msmd1,
                              D=160, HLOC=5, B=40, CHUNKS=(EE // NS) // 40)
    msg1 = jnp.concatenate([msg1_2[:NN], msg1_2[NP:NP + NN]], axis=1)
    den1 = den1_2[:NN]

    h2p, as2v, ad2v, ms2, md2 = _tc2(
        msg1, den1, h, a_s, a_d, ms1, md1, b1.reshape(1, 320), W2,
        As2_m, Ad2_m, R1)

    as2_cat = jnp.concatenate([as2v, as2v], axis=0)
    ad2_cat = jnp.concatenate([ad2v, ad2v], axis=0)
    msmd2 = jnp.concatenate([ms2, ms2, md2, md2], axis=0)
    msg2_2, den2_2 = _sc_edge(h2p, srcE, srcB, dstE, dstB, as2_cat, ad2_cat,
                              msmd2,
                              D=32, HLOC=1,
                              B=200, CHUNKS=(EE // (NC * NS)) // 200)

    out = _tc3(msg2_2[:NN], msg2_2[NP:NP + NN], den2_2[:NN], den2_2[NP:NP + NN],
               h2p, as2v, ad2v, ms2, md2, b2.reshape(1, 32), R2, Wfc,
               bfc.reshape(1, 32))
    return out


# speedup vs baseline: 1.2604x; 1.0590x over previous
import functools
import jax
import jax.numpy as jnp
from jax import lax
from jax.experimental import pallas as pl
from jax.experimental.pallas import tpu as pltpu
from jax.experimental.pallas import tpu_sc as plsc

NEG = 0.2
NN = 10000
EE = 160000
LL = 16
NC = 2
NS = 16
NP = 10240
ROWS_PER_TEC = NP // NS


def _lrelu(x):
    return jnp.where(x > 0, x, NEG * x)


def _tc1_body(x_ref, w1_ref, as_m_ref, ad_m_ref,
              h_ref, as_ref, ad_ref, ms_ref, md_ref):
    i = pl.program_id(0)
    h = jnp.dot(x_ref[...], w1_ref[...], preferred_element_type=jnp.float32)
    h_ref[...] = h
    a_s = jnp.dot(h, as_m_ref[...], preferred_element_type=jnp.float32)
    a_d = jnp.dot(h, ad_m_ref[...], preferred_element_type=jnp.float32)
    as_ref[...] = a_s
    ad_ref[...] = a_d

    @pl.when(i == 0)
    def _():
        ms_ref[...] = jnp.full((1, LL), -1e30, jnp.float32)
        md_ref[...] = jnp.full((1, LL), -1e30, jnp.float32)

    ms_ref[...] = jnp.maximum(ms_ref[...], jnp.max(a_s, axis=0, keepdims=True))
    md_ref[...] = jnp.maximum(md_ref[...], jnp.max(a_d, axis=0, keepdims=True))


def _tc1(x, W1, As_m, Ad_m, block=1000):
    g = NN // block
    return pl.pallas_call(
        _tc1_body,
        grid=(g,),
        in_specs=[
            pl.BlockSpec((block, 128), lambda i: (i, 0)),
            pl.BlockSpec((128, 320), lambda i: (0, 0)),
            pl.BlockSpec((320, LL), lambda i: (0, 0)),
            pl.BlockSpec((320, LL), lambda i: (0, 0)),
        ],
        out_specs=[
            pl.BlockSpec((block, 320), lambda i: (i, 0)),
            pl.BlockSpec((block, LL), lambda i: (i, 0)),
            pl.BlockSpec((block, LL), lambda i: (i, 0)),
            pl.BlockSpec((1, LL), lambda i: (0, 0)),
            pl.BlockSpec((1, LL), lambda i: (0, 0)),
        ],
        out_shape=[
            jax.ShapeDtypeStruct((NN, 320), jnp.float32),
            jax.ShapeDtypeStruct((NN, LL), jnp.float32),
            jax.ShapeDtypeStruct((NN, LL), jnp.float32),
            jax.ShapeDtypeStruct((1, LL), jnp.float32),
            jax.ShapeDtypeStruct((1, LL), jnp.float32),
        ],
    )(x, W1, As_m, Ad_m)


def _subchunks(B):
    subs, off = [], 0
    while off < B:
        sz = min(128, B - off)
        subs.append((off, sz))
        off += sz
    return subs


def _sc_edge_body(D, HLOC, B, CHUNKS, NBUF,
                  hcat, srcA, srcB, dstA, dstB, as_cat, ad_cat, msmd, zmsg,
                  zden,
                  msg_out, den_out,
                  msg_sh, den_sh,
                  src_v, srch_v, dsth_v, rows_v, as_v, ad_v, w_v,
                  msmd_v, dst_subs, sem1, sem2, sem3):
    c = lax.axis_index("c")
    s = lax.axis_index("s")
    subs = _subchunks(B)

    r0 = s * ROWS_PER_TEC
    pltpu.sync_copy(zmsg, msg_sh.at[pl.ds(r0, ROWS_PER_TEC)])
    pltpu.sync_copy(zden, den_sh.at[pl.ds(r0, ROWS_PER_TEC)])

    pltpu.sync_copy(msmd, msmd_v)
    csum0 = msmd_v[0, :] + msmd_v[2, :]
    csum1 = msmd_v[1, :] + msmd_v[3, :]
    csum = jnp.where(c == 0, csum0, csum1)
    cvec = jnp.maximum(_lrelu(csum), 0.0)

    plsc.subcore_barrier()

    if HLOC == 5:
        tec_edges = EE // NS
        edge_base0 = s * tec_edges
    else:
        tec_edges = EE // (NC * NS)
        edge_base0 = c * (EE // NC) + s * tec_edges

    boff = c * EE

    def load_idx(p, k):
        base = edge_base0 + k * B
        pltpu.sync_copy(srcA.at[pl.ds(base, B)], src_v[p])
        pltpu.sync_copy(srcB.at[pl.ds(boff + base, B)], srch_v[p])
        pltpu.sync_copy(dstB.at[pl.ds(boff + base, B)], dsth_v[p])
        for t, (off, sz) in enumerate(subs):
            pltpu.sync_copy(dstA.at[pl.ds(base + off, sz)], dst_subs[p][t])

    def gather_cps(p):
        hidx = srch_v[p] if HLOC == 5 else src_v[p]
        cps = []
        for off, sz in subs:
            cps.append((hcat.at[hidx.at[pl.ds(off, sz)]],
                        rows_v[p].at[pl.ds(off, sz)], sem1[p]))
            cps.append((as_cat.at[srch_v[p].at[pl.ds(off, sz)]],
                        as_v[p].at[pl.ds(off, sz)], sem2[p]))
            cps.append((ad_cat.at[dsth_v[p].at[pl.ds(off, sz)]],
                        ad_v[p].at[pl.ds(off, sz)], sem3[p]))
        return cps

    def fire(p):
        for a, b, sm in gather_cps(p):
            pltpu.async_copy(a, b, sm)

    def drain(p):
        for a, b, sm in gather_cps(p):
            pltpu.make_async_copy(a, b, sm).wait()

    def compute_scatter(p):
        def edge_body(i, _):
            lg = as_v[p][i, :] + ad_v[p][i, :]
            w = jnp.exp(_lrelu(lg) - cvec)
            w_v[p][i, :] = w
            for hh in range(HLOC):
                wsc = w[hh]
                for q in range(2):
                    off = hh * 32 + q * LL
                    rows_v[p][i, pl.ds(off, LL)] = (
                        rows_v[p][i, pl.ds(off, LL)] * wsc)
            return 0

        lax.fori_loop(0, B, edge_body, 0, unroll=2)
        for t, (off, sz) in enumerate(subs):
            pltpu.sync_copy(w_v[p].at[pl.ds(off, sz)],
                            den_sh.at[dst_subs[p][t]], add=True)
            pltpu.sync_copy(rows_v[p].at[pl.ds(off, sz)],
                            msg_sh.at[dst_subs[p][t]], add=True)

    if NBUF == 1:
        def chunk_body(k, _):
            load_idx(0, k)
            fire(0)
            drain(0)
            compute_scatter(0)
            return 0

        lax.fori_loop(0, CHUNKS, chunk_body, 0)
    else:
        pairs = (CHUNKS - 1) // 2
        load_idx(0, 0)
        fire(0)

        def pair_body(j, _):
            load_idx(1, 2 * j + 1)
            fire(1)
            drain(0)
            compute_scatter(0)
            load_idx(0, 2 * j + 2)
            fire(0)
            drain(1)
            compute_scatter(1)
            return 0

        lax.fori_loop(0, pairs, pair_body, 0)
        drain(0)
        compute_scatter(0)
        if CHUNKS % 2 == 0:
            load_idx(1, CHUNKS - 1)
            fire(1)
            drain(1)
            compute_scatter(1)

    plsc.subcore_barrier()

    orow = c * NP + r0
    pltpu.sync_copy(msg_sh.at[pl.ds(r0, ROWS_PER_TEC)],
                    msg_out.at[pl.ds(orow, ROWS_PER_TEC)])
    pltpu.sync_copy(den_sh.at[pl.ds(r0, ROWS_PER_TEC)],
                    den_out.at[pl.ds(orow, ROWS_PER_TEC)])


def _sc_edge(hcat, srcA, srcB, dstA, dstB, as_cat, ad_cat, msmd,
             D, HLOC, B, CHUNKS, NBUF):
    mesh = plsc.VectorSubcoreMesh(core_axis_name="c", subcore_axis_name="s")
    zmsg = jnp.zeros((ROWS_PER_TEC, D), jnp.float32)
    zden = jnp.zeros((ROWS_PER_TEC, LL), jnp.float32)
    body = functools.partial(_sc_edge_body, D, HLOC, B, CHUNKS, NBUF)
    fn = pl.kernel(
        body,
        out_type=[
            jax.ShapeDtypeStruct((NC * NP, D), jnp.float32),
            jax.ShapeDtypeStruct((NC * NP, LL), jnp.float32),
        ],
        mesh=mesh,
        scratch_types=[
            pltpu.VMEM_SHARED((NP, D), jnp.float32),
            pltpu.VMEM_SHARED((NP, LL), jnp.float32),
            [pltpu.VMEM((B,), jnp.int32) for _ in range(NBUF)],
            [pltpu.VMEM((B,), jnp.int32) for _ in range(NBUF)],
            [pltpu.VMEM((B,), jnp.int32) for _ in range(NBUF)],
            [pltpu.VMEM((B, D), jnp.float32) for _ in range(NBUF)],
            [pltpu.VMEM((B, LL), jnp.float32) for _ in range(NBUF)],
            [pltpu.VMEM((B, LL), jnp.float32) for _ in range(NBUF)],
            [pltpu.VMEM((B, LL), jnp.float32) for _ in range(NBUF)],
            pltpu.VMEM((4, LL), jnp.float32),
            [[pltpu.VMEM((sz,), jnp.int32) for _, sz in _subchunks(B)]
             for _ in range(NBUF)],
            [pltpu.SemaphoreType.DMA for _ in range(NBUF)],
            [pltpu.SemaphoreType.DMA for _ in range(NBUF)],
            [pltpu.SemaphoreType.DMA for _ in range(NBUF)],
        ],
        compiler_params=pltpu.CompilerParams(use_tc_tiling_on_sc=False),
    )
    return fn(hcat, srcA, srcB, dstA, dstB, as_cat, ad_cat, msmd, zmsg, zden)


def _tc2_body(msg_ref, den_ref, h_ref, as_ref, ad_ref, ms_ref, md_ref,
              b1_ref, w2_ref, as2m_ref, ad2m_ref, r1_ref,
              h2_ref, as2_ref, ad2_ref, ms2_ref, md2_ref):
    i = pl.program_id(0)
    csum = ms_ref[...] + md_ref[...]
    c1 = jnp.maximum(_lrelu(csum), 0.0)
    sl = _lrelu(as_ref[...] + ad_ref[...])
    wself = jnp.exp(sl - c1)
    dt = den_ref[...] + wself
    wb = jnp.dot(wself, r1_ref[...], preferred_element_type=jnp.float32)
    db = jnp.dot(dt, r1_ref[...], preferred_element_type=jnp.float32)
    msgt = msg_ref[...] + h_ref[...] * wb
    o1 = msgt / jnp.clip(db, 1e-16) + b1_ref[...]
    h1 = jnp.where(o1 > 0, o1, jnp.exp(jnp.minimum(o1, 0.0)) - 1.0)
    h2p = jnp.dot(h1, w2_ref[...], preferred_element_type=jnp.float32)
    h2_ref[...] = h2p
    a_s2 = jnp.dot(h2p, as2m_ref[...], preferred_element_type=jnp.float32)
    a_d2 = jnp.dot(h2p, ad2m_ref[...], preferred_element_type=jnp.float32)
    as2_ref[...] = a_s2
    ad2_ref[...] = a_d2

    @pl.when(i == 0)
    def _():
        ms2_ref[...] = jnp.full((1, LL), -1e30, jnp.float32)
        md2_ref[...] = jnp.full((1, LL), -1e30, jnp.float32)

    ms2_ref[...] = jnp.maximum(ms2_ref[...], jnp.max(a_s2, 0, keepdims=True))
    md2_ref[...] = jnp.maximum(md2_ref[...], jnp.max(a_d2, 0, keepdims=True))


def _tc2(msg1, den1, h, a_s1, a_d1, ms1, md1, b1, W2, As2_m, Ad2_m, R1,
         block=1000):
    g = NN // block
    cst = lambda i: (0, 0)
    blk = lambda i: (i, 0)
    return pl.pallas_call(
        _tc2_body,
        grid=(g,),
        in_specs=[
            pl.BlockSpec((block, 320), blk),
            pl.BlockSpec((block, LL), blk),
            pl.BlockSpec((block, 320), blk),
            pl.BlockSpec((block, LL), blk),
            pl.BlockSpec((block, LL), blk),
            pl.BlockSpec((1, LL), cst),
            pl.BlockSpec((1, LL), cst),
            pl.BlockSpec((1, 320), cst),
            pl.BlockSpec((320, 32), cst),
            pl.BlockSpec((32, LL), cst),
            pl.BlockSpec((32, LL), cst),
            pl.BlockSpec((LL, 320), cst),
        ],
        out_specs=[
            pl.BlockSpec((block, 32), blk),
            pl.BlockSpec((block, LL), blk),
            pl.BlockSpec((block, LL), blk),
            pl.BlockSpec((1, LL), cst),
            pl.BlockSpec((1, LL), cst),
        ],
        out_shape=[
            jax.ShapeDtypeStruct((NN, 32), jnp.float32),
            jax.ShapeDtypeStruct((NN, LL), jnp.float32),
            jax.ShapeDtypeStruct((NN, LL), jnp.float32),
            jax.ShapeDtypeStruct((1, LL), jnp.float32),
            jax.ShapeDtypeStruct((1, LL), jnp.float32),
        ],
    )(msg1, den1, h, a_s1, a_d1, ms1, md1, b1, W2, As2_m, Ad2_m, R1)


def _tc3_body(ma_ref, mb_ref, da_ref, db_ref, h2_ref, as_ref, ad_ref,
              ms_ref, md_ref, b2_ref, r2_ref, wfc_ref, bfc_ref, out_ref):
    i = pl.program_id(0)
    ng = pl.num_programs(0)
    csum = ms_ref[...] + md_ref[...]
    c2 = jnp.maximum(_lrelu(csum), 0.0)
    sl = _lrelu(as_ref[...] + ad_ref[...])
    wself = jnp.exp(sl - c2)
    dt = da_ref[...] + db_ref[...] + wself
    wb = jnp.dot(wself, r2_ref[...], preferred_element_type=jnp.float32)
    dbb = jnp.dot(dt, r2_ref[...], preferred_element_type=jnp.float32)
    msgt = ma_ref[...] + mb_ref[...] + h2_ref[...] * wb
    o2 = msgt / jnp.clip(dbb, 1e-16) + b2_ref[...]
    h2 = jnp.where(o2 > 0, o2, jnp.exp(jnp.minimum(o2, 0.0)) - 1.0)
    m = jnp.max(h2, axis=0, keepdims=True)

    @pl.when(i == 0)
    def _():
        out_ref[...] = jnp.full((1, 32), -1e30, jnp.float32)

    out_ref[...] = jnp.maximum(out_ref[...], m)

    @pl.when(i == ng - 1)
    def _():
        pooled = out_ref[...]
        fc = jnp.dot(pooled, wfc_ref[...],
                     preferred_element_type=jnp.float32) + bfc_ref[...]
        out_ref[...] = jnp.maximum(fc, 0.0)


def _tc3(ma, mb, da, db, h2p, as2, ad2, ms2, md2, b2, R2, Wfc, bfc,
         block=1000):
    g = NN // block
    cst = lambda i: (0, 0)
    blk = lambda i: (i, 0)
    return pl.pallas_call(
        _tc3_body,
        grid=(g,),
        in_specs=[
            pl.BlockSpec((block, 32), blk),
            pl.BlockSpec((block, 32), blk),
            pl.BlockSpec((block, LL), blk),
            pl.BlockSpec((block, LL), blk),
            pl.BlockSpec((block, 32), blk),
            pl.BlockSpec((block, LL), blk),
            pl.BlockSpec((block, LL), blk),
            pl.BlockSpec((1, LL), cst),
            pl.BlockSpec((1, LL), cst),
            pl.BlockSpec((1, 32), cst),
            pl.BlockSpec((LL, 32), cst),
            pl.BlockSpec((32, 32), cst),
            pl.BlockSpec((1, 32), cst),
        ],
        out_specs=pl.BlockSpec((1, 32), cst),
        out_shape=jax.ShapeDtypeStruct((1, 32), jnp.float32),
    )(ma, mb, da, db, h2p, as2, ad2, ms2, md2, b2, R2, Wfc, bfc)


def kernel(x, edge_index, W1, a_s1, a_d1, b1, W2, a_s2, a_d2, b2, Wfc, bfc):
    f32 = jnp.float32
    srcE = edge_index[0].astype(jnp.int32)
    dstE = edge_index[1].astype(jnp.int32)
    srcB = jnp.concatenate([srcE, srcE + NN])
    dstB = jnp.concatenate([dstE, dstE + NN])

    heads320 = jnp.repeat(jnp.arange(10, dtype=jnp.int32), 32)
    As1_m = jnp.zeros((320, LL), f32).at[jnp.arange(320), heads320].set(
        a_s1.reshape(320))
    Ad1_m = jnp.zeros((320, LL), f32).at[jnp.arange(320), heads320].set(
        a_d1.reshape(320))
    As2_m = jnp.zeros((32, LL), f32).at[:, 0].set(a_s2.reshape(32))
    Ad2_m = jnp.zeros((32, LL), f32).at[:, 0].set(a_d2.reshape(32))
    R1 = jnp.zeros((LL, 320), f32).at[heads320, jnp.arange(320)].set(1.0)
    R2 = jnp.zeros((LL, 32), f32).at[0, :].set(1.0)

    h, a_s, a_d, ms1, md1 = _tc1(x, W1, As1_m, Ad1_m)
    shift5 = lambda a: jnp.concatenate(
        [a[:, 5:], jnp.zeros((a.shape[0], 5), f32)], axis=1)
    hcat = jnp.concatenate([h[:, :160], h[:, 160:]], axis=0)
    as_cat = jnp.concatenate([a_s, shift5(a_s)], axis=0)
    ad_cat = jnp.concatenate([a_d, shift5(a_d)], axis=0)
    msmd1 = jnp.concatenate([ms1, shift5(ms1), md1, shift5(md1)], axis=0)
    msg1_2, den1_2 = _sc_edge(hcat, srcE, srcB, dstE, dstB, as_cat, ad_cat,
                              msmd1,
                              D=160, HLOC=5, B=80,
                              CHUNKS=(EE // NS) // 80, NBUF=1)
    msg1 = jnp.concatenate([msg1_2[:NN], msg1_2[NP:NP + NN]], axis=1)
    den1 = den1_2[:NN]

    h2p, as2v, ad2v, ms2, md2 = _tc2(
        msg1, den1, h, a_s, a_d, ms1, md1, b1.reshape(1, 320), W2,
        As2_m, Ad2_m, R1)

    as2_cat = jnp.concatenate([as2v, as2v], axis=0)
    ad2_cat = jnp.concatenate([ad2v, ad2v], axis=0)
    msmd2 = jnp.concatenate([ms2, ms2, md2, md2], axis=0)
    msg2_2, den2_2 = _sc_edge(h2p, srcE, srcB, dstE, dstB, as2_cat, ad2_cat,
                              msmd2,
                              D=32, HLOC=1, B=200,
                              CHUNKS=(EE // (NC * NS)) // 200, NBUF=2)

    out = _tc3(msg2_2[:NN], msg2_2[NP:NP + NN], den2_2[:NN], den2_2[NP:NP + NN],
               h2p, as2v, ad2v, ms2, md2, b2.reshape(1, 32), R2, Wfc,
               bfc.reshape(1, 32))
    return out
